# Initial kernel scaffold; baseline (speedup 1.0000x reference)
import jax
import jax.numpy as jnp
from jax.experimental import pallas as pl


def kernel(feat_src, edge_index, feat_edge, W_prop1, W_prop2, W_asrc, W_adst, W_aedge, scale, offset, W_agg, b_agg, W_applydst, b_applydst, W_apply, b_apply):
    raise NotImplementedError("write your pallas kernel here")


# speedup vs baseline: 2.1465x; 2.1465x over previous
import functools

import jax
import jax.numpy as jnp
from jax import lax
from jax.experimental import pallas as pl
from jax.experimental.pallas import tpu as pltpu
from jax.experimental.pallas import tpu_sc as plsc

N = 10000
E = 320000
D = 128
F = 128
DE = 16
NC, NS = 2, 16
NW = NC * NS
B = 128
NBLK = 79
EPT = B * NBLK
E_PAD = EPT * NW
R0 = 5120
TBL = R0 + 128
ROWS_PT = TBL // NS
BE = 4096
BN = 1000
NEG = -1e30


def _lk(x):
    return jnp.maximum(x, 0.2 * x)


def _clamp_idx(idx_ref, out_ref, lo):
    for kk in range(B // 16):
        sl = pl.ds(kk * 16, 16)
        raw = idx_ref[sl]
        t = raw - lo
        valid = (t >= 0) & (t < R0)
        out_ref[sl] = jnp.where(valid, t, R0 + (raw & 127))


def _pre_nodes_body(x_ref, w1_ref, w2_ref, was_ref, wad_ref,
                    h_ref, as_ref, ad_ref, ma_ref, md_ref):
    x = x_ref[...]
    h = jnp.dot(x, w1_ref[...], preferred_element_type=jnp.float32)
    h_ref[...] = jnp.dot(_lk(h), w2_ref[...],
                         preferred_element_type=jnp.float32)
    asr = jnp.dot(x, was_ref[...], preferred_element_type=jnp.float32)
    ads = jnp.dot(x, wad_ref[...], preferred_element_type=jnp.float32)
    as_ref[...] = asr
    ad_ref[...] = ads
    bm_a = jnp.max(asr, axis=0, keepdims=True)
    bm_d = jnp.max(ads, axis=0, keepdims=True)
    i = pl.program_id(0)

    @pl.when(i == 0)
    def _():
        ma_ref[...] = bm_a
        md_ref[...] = bm_d

    @pl.when(i != 0)
    def _():
        ma_ref[...] = jnp.maximum(ma_ref[...], bm_a)
        md_ref[...] = jnp.maximum(md_ref[...], bm_d)


def _pre_nodes(x, w1, w2, was, wad):
    nf = jax.ShapeDtypeStruct((N, F), jnp.float32)
    return pl.pallas_call(
        _pre_nodes_body,
        grid=(N // BN,),
        in_specs=[
            pl.BlockSpec((BN, D), lambda i: (i, 0)),
            pl.BlockSpec((D, F), lambda i: (0, 0)),
            pl.BlockSpec((F, F), lambda i: (0, 0)),
            pl.BlockSpec((D, F), lambda i: (0, 0)),
            pl.BlockSpec((D, F), lambda i: (0, 0)),
        ],
        out_specs=[pl.BlockSpec((BN, F), lambda i: (i, 0))] * 3
        + [pl.BlockSpec((1, F), lambda i: (0, 0))] * 2,
        out_shape=[nf] * 3 + [jax.ShapeDtypeStruct((1, F), jnp.float32)] * 2,
    )(x, w1, w2, was, wad)


def _pre_edges_body(fe_ref, we_ref, ep_ref, me_ref):
    i = pl.program_id(0)
    ep = jnp.dot(fe_ref[...], we_ref[...], preferred_element_type=jnp.float32)
    rows = i * BE + lax.broadcasted_iota(jnp.int32, (BE, 1), 0)
    ep = jnp.where(rows < E, ep, NEG)
    ep_ref[...] = ep
    bm = jnp.max(ep, axis=0, keepdims=True)

    @pl.when(i == 0)
    def _():
        me_ref[...] = bm

    @pl.when(i != 0)
    def _():
        me_ref[...] = jnp.maximum(me_ref[...], bm)


def _pre_edges(fe_pad, we):
    return pl.pallas_call(
        _pre_edges_body,
        grid=(E_PAD // BE,),
        in_specs=[
            pl.BlockSpec((BE, DE), lambda i: (i, 0)),
            pl.BlockSpec((DE, F), lambda i: (0, 0)),
        ],
        out_specs=[pl.BlockSpec((BE, F), lambda i: (i, 0)),
                   pl.BlockSpec((1, F), lambda i: (0, 0))],
        out_shape=[jax.ShapeDtypeStruct((E_PAD, F), jnp.float32),
                   jax.ShapeDtypeStruct((1, F), jnp.float32)],
    )(fe_pad, we)


def _sc_pass1(asrc, adst, ep, c_v, src3, dst3, zeros):
    mesh = plsc.VectorSubcoreMesh(core_axis_name="c", subcore_axis_name="s")

    @functools.partial(
        pl.kernel,
        out_type=(
            jax.ShapeDtypeStruct((E_PAD, F), jnp.float32),
            jax.ShapeDtypeStruct((NC, TBL, F), jnp.float32),
        ),
        mesh=mesh,
        scratch_types=(
            pltpu.VMEM((B,), jnp.int32),
            pltpu.VMEM((B,), jnp.int32),
            pltpu.VMEM((B,), jnp.int32),
            pltpu.VMEM((B, F), jnp.float32),
            pltpu.VMEM((B, F), jnp.float32),
            pltpu.VMEM((B, F), jnp.float32),
            pltpu.VMEM((B, F), jnp.float32),
            pltpu.VMEM((F,), jnp.float32),
            pltpu.VMEM_SHARED((TBL, F), jnp.float32),
        ),
    )
    def k(a_s, a_d, epr, cv, src_i, dst_i, zz,
          ex_o, sd_o,
          sidx, didx, didx2, abuf, dbuf, ebuf, xbuf, cbuf, sd_s):
        c = lax.axis_index("c")
        s = lax.axis_index("s")
        wid = s * NC + c
        pltpu.sync_copy(zz, sd_s.at[pl.ds(s * ROWS_PT, ROWS_PT)])
        pltpu.sync_copy(cv, cbuf)
        plsc.subcore_barrier()

        cregs = tuple(cbuf[pl.ds(kk * 16, 16)] for kk in range(F // 16))

        def block(j, carry):
            pltpu.sync_copy(src_i.at[wid, j], sidx)
            pltpu.sync_copy(dst_i.at[wid, j], didx)
            pltpu.sync_copy(a_s.at[sidx], abuf)
            pltpu.sync_copy(a_d.at[didx], dbuf)
            base = wid * EPT + j * B
            pltpu.sync_copy(epr.at[pl.ds(base, B)], ebuf)

            def row(i, cs):
                for kk in range(F // 16):
                    sl = pl.ds(kk * 16, 16)
                    v = abuf[i, sl] + dbuf[i, sl] + ebuf[i, sl]
                    v = _lk(v) - cs[kk]
                    xbuf[i, sl] = jnp.exp(v)
                return cs

            cs = lax.fori_loop(0, B, row, carry)
            _clamp_idx(didx, didx2, 0)
            pltpu.sync_copy(xbuf, ex_o.at[pl.ds(base, B)])
            pltpu.sync_copy(xbuf, sd_s.at[didx2], add=True)
            return cs

        lax.fori_loop(0, NBLK, block, cregs)
        plsc.subcore_barrier()
        pltpu.sync_copy(sd_s.at[pl.ds(s * ROWS_PT, ROWS_PT)],
                        sd_o.at[c, pl.ds(s * ROWS_PT, ROWS_PT)])

    return k(asrc, adst, ep, c_v, src3, dst3, zeros)


def _sc_seg(data, idx3, lo, zeros):
    mesh = plsc.VectorSubcoreMesh(core_axis_name="c", subcore_axis_name="s")

    @functools.partial(
        pl.kernel,
        out_type=jax.ShapeDtypeStruct((NC, TBL, F), jnp.float32),
        mesh=mesh,
        scratch_types=(
            pltpu.VMEM((B,), jnp.int32),
            pltpu.VMEM((B,), jnp.int32),
            pltpu.VMEM((B, F), jnp.float32),
            pltpu.VMEM_SHARED((TBL, F), jnp.float32),
        ),
    )
    def k(dat, idx_i, zz, acc_o, idx, idx2, xbuf, acc_s):
        c = lax.axis_index("c")
        s = lax.axis_index("s")
        wid = s * NC + c
        pltpu.sync_copy(zz, acc_s.at[pl.ds(s * ROWS_PT, ROWS_PT)])
        plsc.subcore_barrier()

        def block(j, carry):
            pltpu.sync_copy(idx_i.at[wid, j], idx)
            base = wid * EPT + j * B
            pltpu.sync_copy(dat.at[pl.ds(base, B)], xbuf)
            _clamp_idx(idx, idx2, lo)
            pltpu.sync_copy(xbuf, acc_s.at[idx2], add=True)
            return carry

        lax.fori_loop(0, NBLK, block, 0)
        plsc.subcore_barrier()
        pltpu.sync_copy(acc_s.at[pl.ds(s * ROWS_PT, ROWS_PT)],
                        acc_o.at[c, pl.ds(s * ROWS_PT, ROWS_PT)])

    return k(data, idx3, zeros)


def _combine_body(sd0_ref, sd1_ref, ss0_ref, ss1_ref,
                  rsd0_ref, rsd1_ref, rss0_ref, rss1_ref):
    for p_ref, r_ref in ((sd0_ref, rsd0_ref), (sd1_ref, rsd1_ref),
                         (ss0_ref, rss0_ref), (ss1_ref, rss1_ref)):
        r_ref[...] = lax.rsqrt(jnp.maximum(p_ref[0] + p_ref[1], 1e-20))


def _combine(sd0, sd1, ss0, ss1):
    bc = 1024
    rf = jax.ShapeDtypeStruct((R0, F), jnp.float32)
    return pl.pallas_call(
        _combine_body,
        grid=(R0 // bc,),
        in_specs=[pl.BlockSpec((NC, bc, F), lambda i: (0, i, 0))] * 4,
        out_specs=[pl.BlockSpec((bc, F), lambda i: (i, 0))] * 4,
        out_shape=[rf] * 4,
    )(sd0, sd1, ss0, ss1)


def _sc_pass2(ex, h, rsd, rss, src3, dst3, zeros):
    mesh = plsc.VectorSubcoreMesh(core_axis_name="c", subcore_axis_name="s")

    @functools.partial(
        pl.kernel,
        out_type=(
            jax.ShapeDtypeStruct((E_PAD, F), jnp.float32),
            jax.ShapeDtypeStruct((NC, TBL, F), jnp.float32),
        ),
        mesh=mesh,
        scratch_types=(
            pltpu.VMEM((B,), jnp.int32),
            pltpu.VMEM((B,), jnp.int32),
            pltpu.VMEM((B,), jnp.int32),
            pltpu.VMEM((B, F), jnp.float32),
            pltpu.VMEM((B, F), jnp.float32),
            pltpu.VMEM((B, F), jnp.float32),
            pltpu.VMEM((B, F), jnp.float32),
            pltpu.VMEM_SHARED((TBL, F), jnp.float32),
        ),
    )
    def k(ex_i, h_i, rsd_i, rss_i, src_i, dst_i, zz,
          msg_o, acc_o,
          sidx, didx, didx2, xbuf, hbuf, rdbuf, rsbuf, acc_s):
        c = lax.axis_index("c")
        s = lax.axis_index("s")
        wid = s * NC + c
        pltpu.sync_copy(zz, acc_s.at[pl.ds(s * ROWS_PT, ROWS_PT)])
        plsc.subcore_barrier()

        def block(j, carry):
            pltpu.sync_copy(src_i.at[wid, j], sidx)
            pltpu.sync_copy(dst_i.at[wid, j], didx)
            base = wid * EPT + j * B
            pltpu.sync_copy(ex_i.at[pl.ds(base, B)], xbuf)
            pltpu.sync_copy(h_i.at[sidx], hbuf)
            pltpu.sync_copy(rsd_i.at[didx], rdbuf)
            pltpu.sync_copy(rss_i.at[sidx], rsbuf)

            def row(i, cs):
                for kk in range(F // 16):
                    sl = pl.ds(kk * 16, 16)
                    m = xbuf[i, sl] * rdbuf[i, sl] * rsbuf[i, sl] * hbuf[i, sl]
                    xbuf[i, sl] = m
                return cs

            lax.fori_loop(0, B, row, 0)
            _clamp_idx(didx, didx2, 0)
            pltpu.sync_copy(xbuf, msg_o.at[pl.ds(base, B)])
            pltpu.sync_copy(xbuf, acc_s.at[didx2], add=True)
            return carry

        lax.fori_loop(0, NBLK, block, 0)
        plsc.subcore_barrier()
        pltpu.sync_copy(acc_s.at[pl.ds(s * ROWS_PT, ROWS_PT)],
                        acc_o.at[c, pl.ds(s * ROWS_PT, ROWS_PT)])

    return k(ex, h, rsd, rss, src3, dst3, zeros)


def _post_body(m0_ref, m1_ref, x_ref, wagg_ref, bagg_ref, wad_ref, bad_ref,
               wap_ref, bap_ref, scale_ref, offset_ref, out_ref):
    ms = m0_ref[...] + m1_ref[...]
    mu = jnp.mean(ms, axis=1, keepdims=True)
    xc = ms - mu
    var = jnp.mean(xc * xc, axis=1, keepdims=True) + 1e-9
    hb = xc * scale_ref[...] * lax.rsqrt(var) + offset_ref[...]
    rst = jnp.dot(hb, wagg_ref[...], preferred_element_type=jnp.float32)
    rst = rst + bagg_ref[...]
    rst = rst + jnp.dot(x_ref[...], wad_ref[...],
                        preferred_element_type=jnp.float32) + bad_ref[...]
    rst = _lk(rst)
    out_ref[...] = jnp.dot(rst, wap_ref[...],
                           preferred_element_type=jnp.float32) + bap_ref[...]


def _post(m0, m1, x, wagg, bagg, wad, bad, wap, bap, scale, offset):
    return pl.pallas_call(
        _post_body,
        grid=(N // BN,),
        in_specs=[
            pl.BlockSpec((BN, F), lambda i: (i, 0)),
            pl.BlockSpec((BN, F), lambda i: (i, 0)),
            pl.BlockSpec((BN, D), lambda i: (i, 0)),
            pl.BlockSpec((F, F), lambda i: (0, 0)),
            pl.BlockSpec((1, F), lambda i: (0, 0)),
            pl.BlockSpec((D, F), lambda i: (0, 0)),
            pl.BlockSpec((1, F), lambda i: (0, 0)),
            pl.BlockSpec((F, F), lambda i: (0, 0)),
            pl.BlockSpec((1, F), lambda i: (0, 0)),
            pl.BlockSpec((1, F), lambda i: (0, 0)),
            pl.BlockSpec((1, F), lambda i: (0, 0)),
        ],
        out_specs=pl.BlockSpec((BN, F), lambda i: (i, 0)),
        out_shape=jax.ShapeDtypeStruct((N, F), jnp.float32),
    )(m0, m1, x, wagg, bagg, wad, bad, wap, bap, scale, offset)


def kernel(feat_src, edge_index, feat_edge, W_prop1, W_prop2, W_asrc, W_adst,
           W_aedge, scale, offset, W_agg, b_agg, W_applydst, b_applydst,
           W_apply, b_apply):
    src3 = jnp.pad(edge_index[0], (0, E_PAD - E)).reshape(NW, NBLK, B)
    dst3 = jnp.pad(edge_index[1], (0, E_PAD - E)).reshape(NW, NBLK, B)
    fe_pad = jnp.pad(feat_edge, ((0, E_PAD - E), (0, 0)))

    h, asrc, adst, ma, md = _pre_nodes(feat_src, W_prop1, W_prop2,
                                       W_asrc, W_adst)
    ep, me = _pre_edges(fe_pad, W_aedge)

    c_v = _lk(ma[0] + md[0] + me[0])
    zeros = jnp.zeros((ROWS_PT, F), jnp.float32)

    ex, sd_p0 = _sc_pass1(asrc, adst, ep, c_v, src3, dst3, zeros)
    sd_p1 = _sc_seg(ex, dst3, R0, zeros)
    ss_p0 = _sc_seg(ex, src3, 0, zeros)
    ss_p1 = _sc_seg(ex, src3, R0, zeros)

    rsd0, rsd1, rss0, rss1 = _combine(sd_p0, sd_p1, ss_p0, ss_p1)
    rsd = jnp.concatenate([rsd0, rsd1[:N - R0]], axis=0)
    rss = jnp.concatenate([rss0, rss1[:N - R0]], axis=0)

    msg, op_p0 = _sc_pass2(ex, h, rsd, rss, src3, dst3, zeros)
    op_p1 = _sc_seg(msg, dst3, R0, zeros)

    m0 = jnp.concatenate([op_p0[0, :R0], op_p1[0, :N - R0]], axis=0)
    m1 = jnp.concatenate([op_p0[1, :R0], op_p1[1, :N - R0]], axis=0)

    return _post(m0, m1, feat_src, W_agg, b_agg.reshape(1, F),
                 W_applydst, b_applydst.reshape(1, F),
                 W_apply, b_apply.reshape(1, F),
                 scale.reshape(1, F), offset.reshape(1, F))


# speedup vs baseline: 3.2586x; 1.5181x over previous
import functools

import jax
import jax.numpy as jnp
from jax import lax
from jax.experimental import pallas as pl
from jax.experimental.pallas import tpu as pltpu
from jax.experimental.pallas import tpu_sc as plsc

N = 10000
E = 320000
D = 128
F = 128
DE = 16
NC, NS = 2, 16
NW = NC * NS
B = 128
NBLK = 80
B1 = 64
NBLK1 = 160
EPT = B * NBLK
E_PAD = EPT * NW
R0 = 5120
TBL = R0 + 128
ROWS_PT = TBL // NS
BE = 4096
BN = 1000
NEG = -1e30


def _lk(x):
    return jnp.maximum(x, 0.2 * x)


def _clamp_idx(idx_ref, jj, out_ref, lo, width=B):
    for kk in range(width // 16):
        sl = pl.ds(kk * 16, 16)
        raw = idx_ref[jj, sl]
        t = raw - lo
        valid = (t >= 0) & (t < R0)
        out_ref[sl] = jnp.where(valid, t, R0 + (raw & 127))


def _pre_nodes_body(x_ref, w1_ref, w2_ref, was_ref, wad_ref,
                    h_ref, as_ref, ad_ref, ma_ref, md_ref):
    x = x_ref[...]
    h = jnp.dot(x, w1_ref[...], preferred_element_type=jnp.float32)
    h_ref[...] = jnp.dot(_lk(h), w2_ref[...],
                         preferred_element_type=jnp.float32)
    asr = jnp.dot(x, was_ref[...], preferred_element_type=jnp.float32)
    ads = jnp.dot(x, wad_ref[...], preferred_element_type=jnp.float32)
    as_ref[...] = asr
    ad_ref[...] = ads
    bm_a = jnp.max(asr, axis=0, keepdims=True)
    bm_d = jnp.max(ads, axis=0, keepdims=True)
    i = pl.program_id(0)

    @pl.when(i == 0)
    def _():
        ma_ref[...] = bm_a
        md_ref[...] = bm_d

    @pl.when(i != 0)
    def _():
        ma_ref[...] = jnp.maximum(ma_ref[...], bm_a)
        md_ref[...] = jnp.maximum(md_ref[...], bm_d)


def _pre_nodes(x, w1, w2, was, wad):
    nf = jax.ShapeDtypeStruct((N, F), jnp.float32)
    return pl.pallas_call(
        _pre_nodes_body,
        grid=(N // BN,),
        in_specs=[
            pl.BlockSpec((BN, D), lambda i: (i, 0)),
            pl.BlockSpec((D, F), lambda i: (0, 0)),
            pl.BlockSpec((F, F), lambda i: (0, 0)),
            pl.BlockSpec((D, F), lambda i: (0, 0)),
            pl.BlockSpec((D, F), lambda i: (0, 0)),
        ],
        out_specs=[pl.BlockSpec((BN, F), lambda i: (i, 0))] * 3
        + [pl.BlockSpec((1, F), lambda i: (0, 0))] * 2,
        out_shape=[nf] * 3 + [jax.ShapeDtypeStruct((1, F), jnp.float32)] * 2,
    )(x, w1, w2, was, wad)


def _pre_edges_body(fe_ref, we_ref, ep_ref, me_ref):
    i = pl.program_id(0)
    ep = jnp.dot(fe_ref[...], we_ref[...], preferred_element_type=jnp.float32)
    rows = i * BE + lax.broadcasted_iota(jnp.int32, (BE, 1), 0)
    ep = jnp.where(rows < E, ep, NEG)
    ep_ref[...] = ep
    bm = jnp.max(ep, axis=0, keepdims=True)

    @pl.when(i == 0)
    def _():
        me_ref[...] = bm

    @pl.when(i != 0)
    def _():
        me_ref[...] = jnp.maximum(me_ref[...], bm)


def _pre_edges(fe_pad, we):
    return pl.pallas_call(
        _pre_edges_body,
        grid=(E_PAD // BE,),
        in_specs=[
            pl.BlockSpec((BE, DE), lambda i: (i, 0)),
            pl.BlockSpec((DE, F), lambda i: (0, 0)),
        ],
        out_specs=[pl.BlockSpec((BE, F), lambda i: (i, 0)),
                   pl.BlockSpec((1, F), lambda i: (0, 0))],
        out_shape=[jax.ShapeDtypeStruct((E_PAD, F), jnp.float32),
                   jax.ShapeDtypeStruct((1, F), jnp.float32)],
    )(fe_pad, we)


def _sc_pass1(asrc, adst, ep, c_v, src3, dst3, zeros):
    mesh = plsc.VectorSubcoreMesh(core_axis_name="c", subcore_axis_name="s")
    HB = NBLK1 // 2

    @functools.partial(
        pl.kernel,
        out_type=(
            jax.ShapeDtypeStruct((E_PAD, F), jnp.float32),
            jax.ShapeDtypeStruct((NC, TBL, F), jnp.float32),
        ),
        mesh=mesh,
        scratch_types=(
            pltpu.VMEM((NBLK1 // 2, B1), jnp.int32),
            pltpu.VMEM((NBLK1 // 2, B1), jnp.int32),
            pltpu.VMEM((B1,), jnp.int32),
            pltpu.VMEM((B1, F), jnp.float32),
            pltpu.VMEM((B1, F), jnp.float32),
            pltpu.VMEM((B1, F), jnp.float32),
            pltpu.VMEM((B1, F), jnp.float32),
            pltpu.VMEM((B1, F), jnp.float32),
            pltpu.VMEM((B1, F), jnp.float32),
            pltpu.VMEM((B1, F), jnp.float32),
            pltpu.VMEM((F,), jnp.float32),
            pltpu.VMEM_SHARED((TBL, F), jnp.float32),
            pltpu.SemaphoreType.DMA,
            pltpu.SemaphoreType.DMA,
            pltpu.SemaphoreType.DMA,
        ),
    )
    def k(a_s, a_d, epr, cv, src_i, dst_i, zz,
          ex_o, sd_o,
          sall, dall, d2, ab0, ab1, ab2, db0, db1, db2, xbuf, cbuf, sd_s,
          sg0, sg1, sg2):
        c = lax.axis_index("c")
        s = lax.axis_index("s")
        wid = s * NC + c
        pltpu.sync_copy(zz, sd_s.at[pl.ds(s * ROWS_PT, ROWS_PT)])
        pltpu.sync_copy(cv, cbuf)
        plsc.subcore_barrier()

        cregs = tuple(cbuf[pl.ds(kk * 16, 16)] for kk in range(F // 16))
        sets = ((ab0, db0, sg0), (ab1, db1, sg1), (ab2, db2, sg2))

        def issue(jj, b):
            ab, db, sg = sets[b]
            pltpu.async_copy(a_s.at[sall.at[jj]], ab, sg)
            pltpu.async_copy(a_d.at[dall.at[jj]], db, sg)

        def process(jj, hbase, b, cs):
            ab, db, sg = sets[b]
            pltpu.make_async_copy(a_s.at[sall.at[jj]], ab, sg).wait()
            pltpu.make_async_copy(a_d.at[dall.at[jj]], db, sg).wait()
            base = wid * EPT + (hbase + jj) * B1
            pltpu.sync_copy(epr.at[pl.ds(base, B1)], xbuf)

            def row(i, c_in):
                for rr in range(2):
                    for kk in range(F // 16):
                        sl = pl.ds(kk * 16, 16)
                        v = ab[i * 2 + rr, sl] + db[i * 2 + rr, sl] \
                            + xbuf[i * 2 + rr, sl]
                        v = _lk(v) - c_in[kk]
                        xbuf[i * 2 + rr, sl] = jnp.exp(v)
                return c_in

            cs = lax.fori_loop(0, B1 // 2, row, cs)
            _clamp_idx(dall, jj, d2, 0, B1)
            pltpu.sync_copy(xbuf, sd_s.at[d2], add=True)
            pltpu.sync_copy(xbuf, ex_o.at[pl.ds(base, B1)])
            return cs

        for hh in range(2):
            hbase = hh * HB
            pltpu.sync_copy(src_i.at[wid, pl.ds(hbase, HB)], sall)
            pltpu.sync_copy(dst_i.at[wid, pl.ds(hbase, HB)], dall)
            issue(0, 0)
            issue(1, 1)
            issue(2, 2)

            def triple(t, carry):
                j = t * 3
                for b in (0, 1, 2):
                    jj = j + b
                    carry = process(jj, hbase, b, carry)
                    issue(jj + 3, b)
                return carry

            cregs = lax.fori_loop(0, (HB - 5) // 3, triple, cregs)
            cregs = process(HB - 5, hbase, 0, cregs)
            issue(HB - 2, 0)
            cregs = process(HB - 4, hbase, 1, cregs)
            issue(HB - 1, 1)
            cregs = process(HB - 3, hbase, 2, cregs)
            cregs = process(HB - 2, hbase, 0, cregs)
            cregs = process(HB - 1, hbase, 1, cregs)

        plsc.subcore_barrier()
        pltpu.sync_copy(sd_s.at[pl.ds(s * ROWS_PT, ROWS_PT)],
                        sd_o.at[c, pl.ds(s * ROWS_PT, ROWS_PT)])

    return k(asrc, adst, ep, c_v, src3, dst3, zeros)


def _sc_seg(data, idx3, lo, zeros):
    mesh = plsc.VectorSubcoreMesh(core_axis_name="c", subcore_axis_name="s")

    @functools.partial(
        pl.kernel,
        out_type=jax.ShapeDtypeStruct((NC, TBL, F), jnp.float32),
        mesh=mesh,
        scratch_types=(
            pltpu.VMEM((NBLK, B), jnp.int32),
            pltpu.VMEM((B,), jnp.int32),
            pltpu.VMEM((B, F), jnp.float32),
            pltpu.VMEM((B, F), jnp.float32),
            pltpu.VMEM_SHARED((TBL, F), jnp.float32),
            pltpu.SemaphoreType.DMA,
            pltpu.SemaphoreType.DMA,
        ),
    )
    def k(dat, idx_i, zz, acc_o, iall, d2, xb0, xb1, acc_s, sr0, sr1):
        c = lax.axis_index("c")
        s = lax.axis_index("s")
        wid = s * NC + c
        pltpu.sync_copy(zz, acc_s.at[pl.ds(s * ROWS_PT, ROWS_PT)])
        pltpu.sync_copy(idx_i.at[wid], iall)
        plsc.subcore_barrier()

        sets = ((xb0, sr0), (xb1, sr1))

        def issue(jj, b):
            xb, sr = sets[b]
            pltpu.async_copy(dat.at[pl.ds(wid * EPT + jj * B, B)], xb, sr)

        def process(jj, b):
            xb, sr = sets[b]
            pltpu.make_async_copy(
                dat.at[pl.ds(wid * EPT + jj * B, B)], xb, sr).wait()
            _clamp_idx(iall, jj, d2, lo)
            pltpu.sync_copy(xb, acc_s.at[d2], add=True)

        issue(0, 0)
        issue(1, 1)

        def pair(t, carry):
            j = t * 2
            for b in (0, 1):
                jj = j + b
                process(jj, b)
                issue(jj + 2, b)
            return carry

        lax.fori_loop(0, (NBLK - 2) // 2, pair, 0)
        process(NBLK - 2, 0)
        process(NBLK - 1, 1)
        plsc.subcore_barrier()
        pltpu.sync_copy(acc_s.at[pl.ds(s * ROWS_PT, ROWS_PT)],
                        acc_o.at[c, pl.ds(s * ROWS_PT, ROWS_PT)])

    return k(data, idx3, zeros)


def _combine_body(sd0_ref, sd1_ref, ss0_ref, ss1_ref,
                  rsd0_ref, rsd1_ref, rss0_ref, rss1_ref):
    for p_ref, r_ref in ((sd0_ref, rsd0_ref), (sd1_ref, rsd1_ref),
                         (ss0_ref, rss0_ref), (ss1_ref, rss1_ref)):
        r_ref[...] = lax.rsqrt(jnp.maximum(p_ref[0] + p_ref[1], 1e-20))


def _combine(sd0, sd1, ss0, ss1):
    bc = 1024
    rf = jax.ShapeDtypeStruct((R0, F), jnp.float32)
    return pl.pallas_call(
        _combine_body,
        grid=(R0 // bc,),
        in_specs=[pl.BlockSpec((NC, bc, F), lambda i: (0, i, 0))] * 4,
        out_specs=[pl.BlockSpec((bc, F), lambda i: (i, 0))] * 4,
        out_shape=[rf] * 4,
    )(sd0, sd1, ss0, ss1)


def _scale_h_body(h_ref, rss_ref, h2_ref):
    h2_ref[...] = h_ref[...] * rss_ref[...]


def _scale_h(h, rss):
    return pl.pallas_call(
        _scale_h_body,
        grid=(N // BN,),
        in_specs=[pl.BlockSpec((BN, F), lambda i: (i, 0))] * 2,
        out_specs=pl.BlockSpec((BN, F), lambda i: (i, 0)),
        out_shape=jax.ShapeDtypeStruct((N, F), jnp.float32),
    )(h, rss)


def _sc_pass2(ex, h2, rsd, src3, dst3, zeros):
    mesh = plsc.VectorSubcoreMesh(core_axis_name="c", subcore_axis_name="s")
    HB = NBLK1 // 2

    @functools.partial(
        pl.kernel,
        out_type=(
            jax.ShapeDtypeStruct((E_PAD, F), jnp.float32),
            jax.ShapeDtypeStruct((NC, TBL, F), jnp.float32),
        ),
        mesh=mesh,
        scratch_types=(
            pltpu.VMEM((NBLK1 // 2, B1), jnp.int32),
            pltpu.VMEM((NBLK1 // 2, B1), jnp.int32),
            pltpu.VMEM((B1,), jnp.int32),
            pltpu.VMEM((B1, F), jnp.float32),
            pltpu.VMEM((B1, F), jnp.float32),
            pltpu.VMEM((B1, F), jnp.float32),
            pltpu.VMEM((B1, F), jnp.float32),
            pltpu.VMEM((B1, F), jnp.float32),
            pltpu.VMEM((B1, F), jnp.float32),
            pltpu.VMEM((B1, F), jnp.float32),
            pltpu.VMEM_SHARED((TBL, F), jnp.float32),
            pltpu.SemaphoreType.DMA,
            pltpu.SemaphoreType.DMA,
            pltpu.SemaphoreType.DMA,
        ),
    )
    def k(ex_i, h_i, rsd_i, src_i, dst_i, zz,
          msg_o, acc_o,
          sall, dall, d2, hb0, hb1, hb2, rb0, rb1, rb2, xbuf, acc_s,
          sg0, sg1, sg2):
        c = lax.axis_index("c")
        s = lax.axis_index("s")
        wid = s * NC + c
        pltpu.sync_copy(zz, acc_s.at[pl.ds(s * ROWS_PT, ROWS_PT)])
        plsc.subcore_barrier()

        sets = ((hb0, rb0, sg0), (hb1, rb1, sg1), (hb2, rb2, sg2))

        def issue(jj, b):
            hb, rb, sg = sets[b]
            pltpu.async_copy(h_i.at[sall.at[jj]], hb, sg)
            pltpu.async_copy(rsd_i.at[dall.at[jj]], rb, sg)

        def process(jj, hbase, b, carry):
            hb, rb, sg = sets[b]
            pltpu.make_async_copy(h_i.at[sall.at[jj]], hb, sg).wait()
            pltpu.make_async_copy(rsd_i.at[dall.at[jj]], rb, sg).wait()
            base = wid * EPT + (hbase + jj) * B1
            pltpu.sync_copy(ex_i.at[pl.ds(base, B1)], xbuf)

            def row(i, c_in):
                for rr in range(2):
                    for kk in range(F // 16):
                        sl = pl.ds(kk * 16, 16)
                        ii = i * 2 + rr
                        xbuf[ii, sl] = xbuf[ii, sl] * rb[ii, sl] * hb[ii, sl]
                return c_in

            lax.fori_loop(0, B1 // 2, row, 0)
            _clamp_idx(dall, jj, d2, 0, B1)
            pltpu.sync_copy(xbuf, acc_s.at[d2], add=True)
            pltpu.sync_copy(xbuf, msg_o.at[pl.ds(base, B1)])
            return carry

        for hh in range(2):
            hbase = hh * HB
            pltpu.sync_copy(src_i.at[wid, pl.ds(hbase, HB)], sall)
            pltpu.sync_copy(dst_i.at[wid, pl.ds(hbase, HB)], dall)
            issue(0, 0)
            issue(1, 1)
            issue(2, 2)

            def triple(t, carry):
                j = t * 3
                for b in (0, 1, 2):
                    jj = j + b
                    carry = process(jj, hbase, b, carry)
                    issue(jj + 3, b)
                return carry

            lax.fori_loop(0, (HB - 5) // 3, triple, 0)
            process(HB - 5, hbase, 0, 0)
            issue(HB - 2, 0)
            process(HB - 4, hbase, 1, 0)
            issue(HB - 1, 1)
            process(HB - 3, hbase, 2, 0)
            process(HB - 2, hbase, 0, 0)
            process(HB - 1, hbase, 1, 0)

        plsc.subcore_barrier()
        pltpu.sync_copy(acc_s.at[pl.ds(s * ROWS_PT, ROWS_PT)],
                        acc_o.at[c, pl.ds(s * ROWS_PT, ROWS_PT)])

    return k(ex, h2, rsd, src3, dst3, zeros)


def _post_body(m0_ref, m1_ref, x_ref, wagg_ref, bagg_ref, wad_ref, bad_ref,
               wap_ref, bap_ref, scale_ref, offset_ref, out_ref):
    ms = m0_ref[...] + m1_ref[...]
    mu = jnp.mean(ms, axis=1, keepdims=True)
    xc = ms - mu
    var = jnp.mean(xc * xc, axis=1, keepdims=True) + 1e-9
    hb = xc * scale_ref[...] * lax.rsqrt(var) + offset_ref[...]
    rst = jnp.dot(hb, wagg_ref[...], preferred_element_type=jnp.float32)
    rst = rst + bagg_ref[...]
    rst = rst + jnp.dot(x_ref[...], wad_ref[...],
                        preferred_element_type=jnp.float32) + bad_ref[...]
    rst = _lk(rst)
    out_ref[...] = jnp.dot(rst, wap_ref[...],
                           preferred_element_type=jnp.float32) + bap_ref[...]


def _post(m0, m1, x, wagg, bagg, wad, bad, wap, bap, scale, offset):
    return pl.pallas_call(
        _post_body,
        grid=(N // BN,),
        in_specs=[
            pl.BlockSpec((BN, F), lambda i: (i, 0)),
            pl.BlockSpec((BN, F), lambda i: (i, 0)),
            pl.BlockSpec((BN, D), lambda i: (i, 0)),
            pl.BlockSpec((F, F), lambda i: (0, 0)),
            pl.BlockSpec((1, F), lambda i: (0, 0)),
            pl.BlockSpec((D, F), lambda i: (0, 0)),
            pl.BlockSpec((1, F), lambda i: (0, 0)),
            pl.BlockSpec((F, F), lambda i: (0, 0)),
            pl.BlockSpec((1, F), lambda i: (0, 0)),
            pl.BlockSpec((1, F), lambda i: (0, 0)),
            pl.BlockSpec((1, F), lambda i: (0, 0)),
        ],
        out_specs=pl.BlockSpec((BN, F), lambda i: (i, 0)),
        out_shape=jax.ShapeDtypeStruct((N, F), jnp.float32),
    )(m0, m1, x, wagg, bagg, wad, bad, wap, bap, scale, offset)


def kernel(feat_src, edge_index, feat_edge, W_prop1, W_prop2, W_asrc, W_adst,
           W_aedge, scale, offset, W_agg, b_agg, W_applydst, b_applydst,
           W_apply, b_apply):
    src_pad = jnp.pad(edge_index[0], (0, E_PAD - E))
    dst_pad = jnp.pad(edge_index[1], (0, E_PAD - E))
    src3 = src_pad.reshape(NW, NBLK, B)
    dst3 = dst_pad.reshape(NW, NBLK, B)
    src4 = src_pad.reshape(NW, NBLK1, B1)
    dst4 = dst_pad.reshape(NW, NBLK1, B1)
    fe_pad = jnp.pad(feat_edge, ((0, E_PAD - E), (0, 0)))

    h, asrc, adst, ma, md = _pre_nodes(feat_src, W_prop1, W_prop2,
                                       W_asrc, W_adst)
    ep, me = _pre_edges(fe_pad, W_aedge)

    c_v = _lk(ma[0] + md[0] + me[0])
    zeros = jnp.zeros((ROWS_PT, F), jnp.float32)

    ex, sd_p0 = _sc_pass1(asrc, adst, ep, c_v, src4, dst4, zeros)
    sd_p1 = _sc_seg(ex, dst3, R0, zeros)
    ss_p0 = _sc_seg(ex, src3, 0, zeros)
    ss_p1 = _sc_seg(ex, src3, R0, zeros)

    rsd0, rsd1, rss0, rss1 = _combine(sd_p0, sd_p1, ss_p0, ss_p1)
    rsd = jnp.concatenate([rsd0, rsd1[:N - R0]], axis=0)
    rss = jnp.concatenate([rss0, rss1[:N - R0]], axis=0)
    h2 = _scale_h(h, rss)

    msg, op_p0 = _sc_pass2(ex, h2, rsd, src4, dst4, zeros)
    op_p1 = _sc_seg(msg, dst3, R0, zeros)

    m0 = jnp.concatenate([op_p0[0, :R0], op_p1[0, :N - R0]], axis=0)
    m1 = jnp.concatenate([op_p0[1, :R0], op_p1[1, :N - R0]], axis=0)

    return _post(m0, m1, feat_src, W_agg, b_agg.reshape(1, F),
                 W_applydst, b_applydst.reshape(1, F),
                 W_apply, b_apply.reshape(1, F),
                 scale.reshape(1, F), offset.reshape(1, F))
